# baseline (speedup 1.0000x reference)
import functools

import jax
import jax.numpy as jnp
from jax import lax
from jax.experimental import pallas as pl
from jax.experimental.pallas import tpu as pltpu
from jax.experimental.pallas import tpu_sc as plsc

MAXP = 4096
DEMB = 64
NW = 32
LANES = 16


def kernel(input, embedding):
    S, W = input.shape
    k = min((S - 1) // 2, MAXP)
    rows_w = S // NW
    G = 16
    nt = rows_w // G
    lanes_w = W * DEMB // 128

    mesh = plsc.VectorSubcoreMesh(core_axis_name="c", subcore_axis_name="s")

    @functools.partial(
        pl.kernel,
        mesh=mesh,
        out_type=jax.ShapeDtypeStruct((S * W, DEMB), jnp.float32),
        scratch_types=[
            pltpu.VMEM((rows_w, W), jnp.int32),
            pltpu.VMEM((2, G * W, DEMB), jnp.float32),
            pltpu.SemaphoreType.DMA,
            pltpu.SemaphoreType.DMA,
        ],
        compiler_params=pltpu.CompilerParams(use_tc_tiling_on_sc=False),
    )
    def body(inp_hbm, emb_hbm, out_hbm, idx_v, rows_v, gsem, wsem):
        nc = 2
        wid = lax.axis_index("s") * nc + lax.axis_index("c")
        s0 = wid * rows_w

        pltpu.sync_copy(inp_hbm.at[pl.ds(s0, rows_w)], idx_v)

        def transform(t):
            def fix(r, c):
                for q in range(W // LANES):
                    v = idx_v[r, pl.ds(q * LANES, LANES)]
                    v = jnp.clip(v, -MAXP, MAXP) + k
                    idx_v[r, pl.ds(q * LANES, LANES)] = v
                return c

            lax.fori_loop(t * G, (t + 1) * G, fix, 0)

        def fire_gathers(t, p):
            for a in range(G):
                pltpu.async_copy(
                    emb_hbm.at[idx_v.at[t * G + a]],
                    rows_v.at[p, pl.ds(a * W, W)], gsem,
                )

        def drain_gathers():
            for a in range(G):
                pltpu.make_async_copy(
                    emb_hbm.at[idx_v.at[0]],
                    rows_v.at[0, pl.ds(a * W, W)], gsem,
                ).wait()

        transform(0)
        fire_gathers(0, 0)

        def step(t, carry):
            p = lax.rem(t, 2)

            @pl.when(t + 1 < nt)
            def _():
                transform(t + 1)

            drain_gathers()

            @pl.when(t >= 1)
            def _():
                pltpu.make_async_copy(
                    rows_v.at[0], out_hbm.at[pl.ds(0, G * W)], wsem
                ).wait()

            @pl.when(t + 1 < nt)
            def _():
                fire_gathers(t + 1, 1 - p)

            pltpu.async_copy(
                rows_v.at[p],
                out_hbm.at[pl.ds((s0 + t * G) * W, G * W)],
                wsem,
            )
            return carry

        lax.fori_loop(0, nt, step, 0)
        pltpu.make_async_copy(
            rows_v.at[0], out_hbm.at[pl.ds(0, G * W)], wsem
        ).wait()

    out = body(input.astype(jnp.int32), embedding)
    return out.reshape(S, W, DEMB)


# speedup vs baseline: 1.0040x; 1.0040x over previous
import functools

import jax
import jax.numpy as jnp
from jax import lax
from jax.experimental import pallas as pl
from jax.experimental.pallas import tpu as pltpu
from jax.experimental.pallas import tpu_sc as plsc

MAXP = 4096
DEMB = 64
NW = 32
LANES = 16


def kernel(input, embedding):
    S, W = input.shape
    k = min((S - 1) // 2, MAXP)
    rows_w = S // NW
    G = 16
    nt = rows_w // G
    lanes_w = W * DEMB // 128

    mesh = plsc.VectorSubcoreMesh(core_axis_name="c", subcore_axis_name="s")

    @functools.partial(
        pl.kernel,
        mesh=mesh,
        out_type=jax.ShapeDtypeStruct((S * W, DEMB), jnp.float32),
        scratch_types=[
            pltpu.VMEM((rows_w, W), jnp.int32),
            pltpu.VMEM((2, G * W, DEMB), jnp.float32),
            pltpu.SemaphoreType.DMA,
            pltpu.SemaphoreType.DMA,
            pltpu.SemaphoreType.DMA,
            pltpu.SemaphoreType.DMA,
        ],
        compiler_params=pltpu.CompilerParams(use_tc_tiling_on_sc=False),
    )
    def body(inp_hbm, emb_hbm, out_hbm, idx_v, rows_v, g0, g1, w0, w1):
        nc = 2
        wid = lax.axis_index("s") * nc + lax.axis_index("c")
        s0 = wid * rows_w

        pltpu.sync_copy(inp_hbm.at[pl.ds(s0, rows_w)], idx_v)

        def transform(t):
            def fix(r, c):
                for q in range(W // LANES):
                    v = idx_v[r, pl.ds(q * LANES, LANES)]
                    v = jnp.clip(v, -MAXP, MAXP) + k
                    idx_v[r, pl.ds(q * LANES, LANES)] = v
                return c

            lax.fori_loop(t * G, (t + 1) * G, fix, 0)

        def fire_gathers(t, p, sem):
            for a in range(G):
                pltpu.async_copy(
                    emb_hbm.at[idx_v.at[t * G + a]],
                    rows_v.at[p, pl.ds(a * W, W)], sem,
                )

        def drain_gathers(sem):
            for a in range(G):
                pltpu.make_async_copy(
                    emb_hbm.at[idx_v.at[0]],
                    rows_v.at[0, pl.ds(a * W, W)], sem,
                ).wait()

        def wait_write(sem):
            pltpu.make_async_copy(
                rows_v.at[0], out_hbm.at[pl.ds(0, G * W)], sem
            ).wait()

        transform(0)
        fire_gathers(0, 0, g0)

        def step2(t2, carry):
            for half, pb, gs_cur, gs_nxt, ws_cur, ws_nxt in (
                (0, 0, g0, g1, w0, w1),
                (1, 1, g1, g0, w1, w0),
            ):
                t = 2 * t2 + half

                @pl.when(t + 1 < nt)
                def _():
                    transform(t + 1)

                @pl.when(t >= 1)
                def _():
                    wait_write(ws_nxt)

                @pl.when(t + 1 < nt)
                def _():
                    fire_gathers(t + 1, 1 - pb, gs_nxt)

                drain_gathers(gs_cur)

                pltpu.async_copy(
                    rows_v.at[pb],
                    out_hbm.at[pl.ds((s0 + t * G) * W, G * W)],
                    ws_cur,
                )
            return carry

        lax.fori_loop(0, nt // 2, step2, 0)
        wait_write(w1)

    out = body(input.astype(jnp.int32), embedding)
    return out.reshape(S, W, DEMB)


# speedup vs baseline: 1.0060x; 1.0021x over previous
import functools

import jax
import jax.numpy as jnp
from jax import lax
from jax.experimental import pallas as pl
from jax.experimental.pallas import tpu as pltpu
from jax.experimental.pallas import tpu_sc as plsc

MAXP = 4096
DEMB = 64
NW = 32
LANES = 16


def kernel(input, embedding):
    S, W = input.shape
    k = min((S - 1) // 2, MAXP)
    rows_w = S // NW
    G = 16
    nt = rows_w // G
    lanes_w = W * DEMB // 128

    mesh = plsc.VectorSubcoreMesh(core_axis_name="c", subcore_axis_name="s")

    @functools.partial(
        pl.kernel,
        mesh=mesh,
        out_type=jax.ShapeDtypeStruct((S * W, DEMB), jnp.float32),
        scratch_types=[
            pltpu.VMEM((rows_w, W), jnp.int32),
            pltpu.VMEM((2, G * W, DEMB), jnp.float32),
            pltpu.SemaphoreType.DMA,
            pltpu.SemaphoreType.DMA,
            pltpu.SemaphoreType.DMA,
            pltpu.SemaphoreType.DMA,
        ],
        compiler_params=pltpu.CompilerParams(use_tc_tiling_on_sc=False),
    )
    def body(inp_hbm, emb_hbm, out_hbm, idx_v, rows_v, g0, g1, w0, w1):
        nc = 2
        wid = lax.axis_index("s") * nc + lax.axis_index("c")
        s0 = wid * rows_w

        pltpu.sync_copy(inp_hbm.at[pl.ds(s0, rows_w)], idx_v)

        def transform(t):
            def fix(r, c):
                for q in range(W // LANES):
                    v = idx_v[r, pl.ds(q * LANES, LANES)]
                    v = jnp.maximum(jnp.clip(v, -MAXP, MAXP) + k, 0)
                    idx_v[r, pl.ds(q * LANES, LANES)] = v
                return c

            lax.fori_loop(t * G, (t + 1) * G, fix, 0)

        def fire_gathers(t, p, sem):
            for a in range(G):
                pltpu.async_copy(
                    emb_hbm.at[idx_v.at[t * G + a]],
                    rows_v.at[p, pl.ds(a * W, W)], sem,
                )

        def drain_gathers(sem):
            for a in range(G):
                pltpu.make_async_copy(
                    emb_hbm.at[idx_v.at[0]],
                    rows_v.at[0, pl.ds(a * W, W)], sem,
                ).wait()

        def wait_write(sem):
            pltpu.make_async_copy(
                rows_v.at[0], out_hbm.at[pl.ds(0, G * W)], sem
            ).wait()

        transform(0)
        fire_gathers(0, 0, g0)

        def step2(t2, carry):
            for half, pb, gs_cur, gs_nxt, ws_cur, ws_nxt in (
                (0, 0, g0, g1, w0, w1),
                (1, 1, g1, g0, w1, w0),
            ):
                t = 2 * t2 + half

                @pl.when(t + 1 < nt)
                def _():
                    transform(t + 1)

                @pl.when(t >= 1)
                def _():
                    wait_write(ws_nxt)

                @pl.when(t + 1 < nt)
                def _():
                    fire_gathers(t + 1, 1 - pb, gs_nxt)

                drain_gathers(gs_cur)

                pltpu.async_copy(
                    rows_v.at[pb],
                    out_hbm.at[pl.ds((s0 + t * G) * W, G * W)],
                    ws_cur,
                )
            return carry

        lax.fori_loop(0, nt // 2, step2, 0)
        wait_write(w1)

    out = body(input.astype(jnp.int32), embedding)
    return out.reshape(S, W, DEMB)


# speedup vs baseline: 1.6840x; 1.6739x over previous
import functools

import jax
import jax.numpy as jnp
from jax import lax
from jax.experimental import pallas as pl
from jax.experimental.pallas import tpu as pltpu
from jax.experimental.pallas import tpu_sc as plsc

MAXP = 4096
DEMB = 64
NW = 32
LANES = 16


def kernel(input, embedding):
    S, W = input.shape
    k = min((S - 1) // 2, MAXP)
    rows_w = S // NW
    G = 16
    nt = rows_w // G
    lanes_w = W * DEMB // 128

    mesh = plsc.VectorSubcoreMesh(core_axis_name="c", subcore_axis_name="s")

    @functools.partial(
        pl.kernel,
        mesh=mesh,
        out_type=jax.ShapeDtypeStruct((S, W, 2 * DEMB), jnp.float32),
        scratch_types=[
            pltpu.VMEM((rows_w, W), jnp.int32),
            pltpu.VMEM((2, G, W, DEMB), jnp.float32),
            pltpu.SemaphoreType.DMA,
            pltpu.SemaphoreType.DMA,
            pltpu.SemaphoreType.DMA,
            pltpu.SemaphoreType.DMA,
        ],
        compiler_params=pltpu.CompilerParams(use_tc_tiling_on_sc=False),
    )
    def body(inp_hbm, emb_hbm, out_hbm, idx_v, rows_v, g0, g1, w0, w1):
        nc = 2
        wid = lax.axis_index("s") * nc + lax.axis_index("c")
        s0 = wid * rows_w

        pltpu.sync_copy(inp_hbm.at[pl.ds(s0, rows_w)], idx_v)

        def transform(t):
            def fix(r, c):
                for q in range(W // LANES):
                    v = idx_v[r, pl.ds(q * LANES, LANES)]
                    v = jnp.maximum(jnp.clip(v, -MAXP, MAXP) + k, 0)
                    idx_v[r, pl.ds(q * LANES, LANES)] = v
                return c

            lax.fori_loop(t * G, (t + 1) * G, fix, 0)

        def fire_gathers(t, p, sem):
            for a in range(G):
                pltpu.async_copy(
                    emb_hbm.at[idx_v.at[t * G + a]],
                    rows_v.at[p, a], sem,
                )

        def drain_gathers(sem):
            for a in range(G):
                pltpu.make_async_copy(
                    emb_hbm.at[idx_v.at[0]],
                    rows_v.at[0, 0], sem,
                ).wait()

        def wait_write(sem):
            pltpu.make_async_copy(
                rows_v.at[0],
                out_hbm.at[pl.ds(0, G), :, pl.ds(0, DEMB)], sem
            ).wait()

        transform(0)
        fire_gathers(0, 0, g0)

        def step2(t2, carry):
            for half, pb, gs_cur, gs_nxt, ws_cur, ws_nxt in (
                (0, 0, g0, g1, w0, w1),
                (1, 1, g1, g0, w1, w0),
            ):
                t = 2 * t2 + half

                @pl.when(t + 1 < nt)
                def _():
                    transform(t + 1)

                @pl.when(t >= 1)
                def _():
                    wait_write(ws_nxt)

                @pl.when(t + 1 < nt)
                def _():
                    fire_gathers(t + 1, 1 - pb, gs_nxt)

                drain_gathers(gs_cur)

                pltpu.async_copy(
                    rows_v.at[pb],
                    out_hbm.at[pl.ds(s0 + t * G, G), :, pl.ds(0, DEMB)],
                    ws_cur,
                )
            return carry

        lax.fori_loop(0, nt // 2, step2, 0)
        wait_write(w1)

    out = body(input.astype(jnp.int32), embedding)
    return out[:, :, :DEMB]


# speedup vs baseline: 1.6886x; 1.0027x over previous
import functools

import jax
import jax.numpy as jnp
from jax import lax
from jax.experimental import pallas as pl
from jax.experimental.pallas import tpu as pltpu
from jax.experimental.pallas import tpu_sc as plsc

MAXP = 4096
DEMB = 64
NW = 32
LANES = 16


def kernel(input, embedding):
    S, W = input.shape
    k = min((S - 1) // 2, MAXP)
    flat_w = S * W // NW
    GW = 512
    nt = flat_w // GW

    mesh = plsc.VectorSubcoreMesh(core_axis_name="c", subcore_axis_name="s")

    @functools.partial(
        pl.kernel,
        mesh=mesh,
        out_type=jax.ShapeDtypeStruct((S * W, 2 * DEMB), jnp.float32),
        scratch_types=[
            pltpu.VMEM((flat_w,), jnp.int32),
            pltpu.VMEM((2, GW, DEMB), jnp.float32),
            pltpu.SemaphoreType.DMA,
            pltpu.SemaphoreType.DMA,
            pltpu.SemaphoreType.DMA,
            pltpu.SemaphoreType.DMA,
        ],
        compiler_params=pltpu.CompilerParams(use_tc_tiling_on_sc=False),
    )
    def body(inp_hbm, emb_hbm, out_hbm, idx_v, rows_v, g0, g1, w0, w1):
        nc = 2
        wid = lax.axis_index("s") * nc + lax.axis_index("c")
        f0 = wid * flat_w

        pltpu.sync_copy(inp_hbm.at[pl.ds(f0, flat_w)], idx_v)

        def transform(t):
            def fix(c, carry):
                v = idx_v[pl.ds(c * LANES, LANES)]
                v = jnp.maximum(jnp.clip(v, -MAXP, MAXP) + k, 0)
                idx_v[pl.ds(c * LANES, LANES)] = v
                return carry

            lax.fori_loop(t * GW // LANES, (t + 1) * GW // LANES, fix, 0)

        def fire_gather(t, p, sem):
            pltpu.async_copy(
                emb_hbm.at[idx_v.at[pl.ds(t * GW, GW)]],
                rows_v.at[p], sem,
            )

        def drain_gather(sem):
            pltpu.make_async_copy(
                emb_hbm.at[idx_v.at[pl.ds(0, GW)]],
                rows_v.at[0], sem,
            ).wait()

        def wait_write(sem):
            pltpu.make_async_copy(
                rows_v.at[0],
                out_hbm.at[pl.ds(0, GW), pl.ds(0, DEMB)], sem
            ).wait()

        transform(0)
        fire_gather(0, 0, g0)

        def step2(t2, carry):
            for half, pb, gs_cur, gs_nxt, ws_cur, ws_nxt in (
                (0, 0, g0, g1, w0, w1),
                (1, 1, g1, g0, w1, w0),
            ):
                t = 2 * t2 + half

                @pl.when(t + 1 < nt)
                def _():
                    transform(t + 1)

                @pl.when(t >= 1)
                def _():
                    wait_write(ws_nxt)

                @pl.when(t + 1 < nt)
                def _():
                    fire_gather(t + 1, 1 - pb, gs_nxt)

                drain_gather(gs_cur)

                pltpu.async_copy(
                    rows_v.at[pb],
                    out_hbm.at[pl.ds(f0 + t * GW, GW), pl.ds(0, DEMB)],
                    ws_cur,
                )
            return carry

        lax.fori_loop(0, nt // 2, step2, 0)
        wait_write(w1)

    out = body(input.astype(jnp.int32).reshape(S * W), embedding)
    return out.reshape(S, W, 2 * DEMB)[:, :, :DEMB]
